# Initial kernel scaffold; baseline (speedup 1.0000x reference)
import jax
import jax.numpy as jnp
from jax.experimental import pallas as pl


def kernel(_xyz, _scaling, _rotation, _features_dc, _opacity):
    raise NotImplementedError("write your pallas kernel here")


# speedup vs baseline: 3.4799x; 3.4799x over previous
import functools

import jax
import jax.numpy as jnp
import numpy as np
from jax import lax
from jax.experimental import pallas as pl
from jax.experimental.pallas import tpu as pltpu
from jax.experimental.pallas import tpu_sc as plsc

N = 50000
H = 512
W = 512
R = 6
K = 2 * R + 1
KK = K * K

G = 512
NPAD = 53248
NB = NPAD // G
TOT = KK * NPAD
HWPIX = H * W

NWORKERS = 32
TOT3 = 3 * TOT
IDX_ROWS = TOT3 // 128
ROWS_PER_TILE = IDX_ROWS // NWORKERS
NCHUNK = 39
CHUNK_ROWS = ROWS_PER_TILE // NCHUNK
CHUNK_E = CHUNK_ROWS * 128
EPT = ROWS_PER_TILE * 128
FBW = HWPIX * 3
FB_SLICE = FBW // 16
ZCHUNK = 16384


def _splat_body(xyz_ref, scl_ref, rot_ref, feat_ref, opac_ref,
                fr_ref, fg_ref, fb_ref, cr_ref, cg_ref, cb_ref):
    r = pl.program_id(1)
    xy = jnp.tanh(xyz_ref[...])
    px = (0.5 * W) * (xy[:, 0] + 1.0)
    py = (0.5 * H) * (xy[:, 1] + 1.0)
    scl = jnp.abs(scl_ref[...] + 0.5)
    s1 = scl[:, 0]
    s2 = scl[:, 1]
    rot = jax.nn.sigmoid(rot_ref[:, 0]) * (2.0 * np.pi)
    c = jnp.cos(rot)
    s = jnp.sin(rot)
    S00 = s1 * s1 * c * c + s2 * s2 * s * s
    S01 = (s1 * s1 - s2 * s2) * s * c
    S11 = s1 * s1 * s * s + s2 * s2 * c * c
    det = jnp.maximum(S00 * S11 - S01 * S01, 1e-12)
    A = (S11 / det)[None, :]
    B = (-S01 / det)[None, :]
    C = (S00 / det)[None, :]
    cx = jnp.floor(px).astype(jnp.int32)[None, :]
    cy = jnp.floor(py).astype(jnp.int32)[None, :]
    ox = lax.broadcasted_iota(jnp.int32, (K, 1), 0) - R
    oy = r - R
    xi = cx + ox
    yi = cy + oy
    dx = xi.astype(jnp.float32) + 0.5 - px[None, :]
    dy = yi.astype(jnp.float32) + 0.5 - py[None, :]
    sigma = 0.5 * (A * dx * dx + C * dy * dy) + B * dx * dy
    alpha = jnp.where(sigma >= 0.0, jnp.exp(-sigma), 0.0) * opac_ref[:, 0][None, :]
    valid = (xi >= 0) & (xi < W) & (yi >= 0) & (yi < H)
    alpha = jnp.where(valid, alpha, 0.0)
    flat3 = 3 * (jnp.clip(yi, 0, H - 1) * W
                 + jnp.clip(xi, 0, W - 1)).astype(jnp.int32)
    fr_ref[...] = flat3[None]
    fg_ref[...] = (flat3 + 1)[None]
    fb_ref[...] = (flat3 + 2)[None]
    f = feat_ref[...]
    cr_ref[...] = (alpha * f[:, 0][None, :])[None]
    cg_ref[...] = (alpha * f[:, 1][None, :])[None]
    cb_ref[...] = (alpha * f[:, 2][None, :])[None]


def _splat_call(xyz, scl, rot, feat, opac):
    out6 = [jax.ShapeDtypeStruct((K, K, NPAD), jnp.int32)] * 3 + \
           [jax.ShapeDtypeStruct((K, K, NPAD), jnp.float32)] * 3
    in_spec = lambda bs: pl.BlockSpec(bs, lambda i, r: (i, 0))
    out_spec = pl.BlockSpec((1, K, G), lambda i, r: (r, 0, i))
    return pl.pallas_call(
        _splat_body,
        grid=(NB, K),
        in_specs=[in_spec((G, 2)), in_spec((G, 2)), in_spec((G, 1)),
                  in_spec((G, 3)), in_spec((G, 1))],
        out_specs=[out_spec] * 6,
        out_shape=out6,
    )(xyz, scl, rot, feat, opac)


def _sc_scatter_body(flat_hbm, contrib_hbm, zeros_hbm, out_hbm,
                     idx_v, cbuf, fb):
    c = lax.axis_index("c")
    s = lax.axis_index("s")
    wid = c * 16 + s
    pltpu.sync_copy(zeros_hbm, cbuf.at[pl.ds(0, ZCHUNK)])

    def zero(i, carry):
        pltpu.sync_copy(cbuf.at[pl.ds(0, ZCHUNK)],
                        fb.at[pl.ds(s * FB_SLICE + i * ZCHUNK, ZCHUNK)])
        return carry

    lax.fori_loop(0, FB_SLICE // ZCHUNK, zero, 0)
    plsc.subcore_barrier()

    def chunk(it, carry):
        g = wid * NCHUNK + it
        e0 = wid * EPT + it * CHUNK_E
        pltpu.sync_copy(flat_hbm.at[g], idx_v)
        pltpu.sync_copy(contrib_hbm.at[pl.ds(e0, CHUNK_E)], cbuf)

        def scat(j, carry2):
            pltpu.sync_copy(cbuf.at[pl.ds(j * 128, 128)],
                            fb.at[idx_v.at[j]], add=True)
            return carry2

        return lax.fori_loop(0, CHUNK_ROWS, scat, carry)

    lax.fori_loop(0, NCHUNK, chunk, 0)
    plsc.subcore_barrier()
    pltpu.sync_copy(fb.at[pl.ds(s * FB_SLICE, FB_SLICE)],
                    out_hbm.at[c, pl.ds(s * FB_SLICE, FB_SLICE)])


def _sc_scatter(flat1, contrib, zeros):
    mesh = plsc.VectorSubcoreMesh(core_axis_name="c", subcore_axis_name="s")
    fn = pl.kernel(
        _sc_scatter_body,
        out_type=jax.ShapeDtypeStruct((2, FBW), jnp.float32),
        mesh=mesh,
        scratch_types=[
            pltpu.VMEM((CHUNK_ROWS, 128), jnp.int32),
            pltpu.VMEM((CHUNK_E,), jnp.float32),
            pltpu.VMEM_SHARED((FBW,), jnp.float32),
        ],
        compiler_params=pltpu.CompilerParams(use_tc_tiling_on_sc=False),
    )
    return fn(flat1.reshape(NWORKERS * NCHUNK, CHUNK_ROWS, 128),
              contrib, zeros)


def _combine_body(fb_ref, out_ref):
    out_ref[...] = jnp.clip(fb_ref[0, :] + fb_ref[1, :], 0.0, 1.0)


def _combine_call(fb2):
    blk = HWPIX * 3 // 8
    return pl.pallas_call(
        _combine_body,
        grid=(8,),
        in_specs=[pl.BlockSpec((2, blk), lambda i: (0, i))],
        out_specs=pl.BlockSpec((blk,), lambda i: (i,)),
        out_shape=jax.ShapeDtypeStruct((HWPIX * 3,), jnp.float32),
    )(fb2)


def _pad_inputs(xyz, scl, rot, feat, opac):
    p = NPAD - N
    t = (jnp.arange(p, dtype=jnp.float32) + 0.5) / p
    tx = jnp.arctanh(t * 1.98 - 0.99)
    ty = jnp.arctanh(((t * 37.0) % 1.0) * 1.98 - 0.99)
    xyz_p = jnp.concatenate([xyz, jnp.stack([tx, ty], axis=-1)], axis=0)
    scl_p = jnp.concatenate([scl, jnp.zeros((p, 2), jnp.float32)], axis=0)
    rot_p = jnp.concatenate([rot, jnp.zeros((p, 1), jnp.float32)], axis=0)
    feat_p = jnp.concatenate([feat, jnp.zeros((p, 3), jnp.float32)], axis=0)
    opac_p = jnp.concatenate([opac, jnp.zeros((p, 1), jnp.float32)], axis=0)
    return xyz_p, scl_p, rot_p, feat_p, opac_p


def kernel(_xyz, _scaling, _rotation, _features_dc, _opacity):
    xyz, scl, rot, feat, opac = _pad_inputs(
        _xyz, _scaling, _rotation, _features_dc, _opacity)
    f0, f1, f2, cr, cg, cb = _splat_call(xyz, scl, rot, feat, opac)
    flat1 = jnp.concatenate(
        [f0.reshape(TOT), f1.reshape(TOT), f2.reshape(TOT)])
    contrib = jnp.concatenate(
        [cr.reshape(TOT), cg.reshape(TOT), cb.reshape(TOT)])
    zeros = jnp.zeros((ZCHUNK,), jnp.float32)
    fb2 = _sc_scatter(flat1, contrib, zeros)
    img = _combine_call(fb2)
    return img.reshape(1, H, W, 3).transpose(0, 3, 1, 2)


# speedup vs baseline: 20.1147x; 5.7802x over previous
import functools

import jax
import jax.numpy as jnp
import numpy as np
from jax import lax
from jax.experimental import pallas as pl
from jax.experimental.pallas import tpu as pltpu
from jax.experimental.pallas import tpu_sc as plsc

N = 50000
H = 512
W = 512
R = 6
K = 2 * R + 1

G = 512
NP = 50176
NB = NP // G

PCH = 12544
LCAP = 296
CAP = 16 * LCAP
CAPR = CAP // 128
BANDH = 16
FBN = BANDH * W


def _prep_body(xyz_ref, scl_ref, rot_ref, feat_ref, opac_ref,
               px_ref, py_ref, a_ref, b_ref, c_ref,
               fr_ref, fg_ref, fb_ref):
    xy = jnp.tanh(xyz_ref[...])
    px_ref[...] = (0.5 * W) * (xy[:, 0] + 1.0)
    py_ref[...] = (0.5 * H) * (xy[:, 1] + 1.0)
    scl = jnp.abs(scl_ref[...] + 0.5)
    s1 = scl[:, 0]
    s2 = scl[:, 1]
    rot = jax.nn.sigmoid(rot_ref[:, 0]) * (2.0 * np.pi)
    c = jnp.cos(rot)
    s = jnp.sin(rot)
    S00 = s1 * s1 * c * c + s2 * s2 * s * s
    S01 = (s1 * s1 - s2 * s2) * s * c
    S11 = s1 * s1 * s * s + s2 * s2 * c * c
    det = jnp.maximum(S00 * S11 - S01 * S01, 1e-12)
    a_ref[...] = S11 / det
    b_ref[...] = -S01 / det
    c_ref[...] = S00 / det
    f = feat_ref[...] * opac_ref[...]
    fr_ref[...] = f[:, 0]
    fg_ref[...] = f[:, 1]
    fb_ref[...] = f[:, 2]


def _prep_call(xyz, scl, rot, feat, opac):
    outs = [jax.ShapeDtypeStruct((NP,), jnp.float32)] * 8
    in_spec = lambda bs: pl.BlockSpec(bs, lambda i: (i, 0))
    out_spec = pl.BlockSpec((G,), lambda i: (i,))
    return pl.pallas_call(
        _prep_body,
        grid=(NB,),
        in_specs=[in_spec((G, 2)), in_spec((G, 2)), in_spec((G, 1)),
                  in_spec((G, 3)), in_spec((G, 1))],
        out_specs=[out_spec] * 8,
        out_shape=outs,
    )(xyz, scl, rot, feat, opac)


def _band_body(px_hbm, py_hbm, a_hbm, b_hbm, c_hbm, fr_hbm, fg_hbm, fb_hbm,
               out_hbm,
               pybuf, idx2, pxb, pyb, ab, bb, cb, frb, fgb, fbb,
               im_r, im_g, im_b):
    cidx = lax.axis_index("c")
    sidx = lax.axis_index("s")
    wid = cidx * 16 + sidx
    r0 = wid * BANDH
    i16 = lax.iota(jnp.int32, 16)
    z16f = jnp.zeros((16,), jnp.float32)
    z16i = jnp.zeros((16,), jnp.int32)

    def zero(i, carry):
        im_r[pl.ds(i * 16, 16)] = z16f
        im_g[pl.ds(i * 16, 16)] = z16f
        im_b[pl.ds(i * 16, 16)] = z16f
        return carry

    lax.fori_loop(0, FBN // 16, zero, 0)

    def prefill(i, carry):
        idx2[pl.ds(i * 16, 16)] = z16i
        return carry

    lax.fori_loop(0, CAP // 16, prefill, 0)

    one16 = jnp.full((16,), 1, jnp.int32)
    lane_base = i16 * LCAP

    def chunk_scan(ci, clv0):
        pltpu.sync_copy(py_hbm.at[pl.ds(ci * PCH, PCH)], pybuf)

        def grp(i, clv):
            pyv = pybuf[pl.ds(i * 16, 16)]
            cyv = pyv.astype(jnp.int32)
            m = (cyv >= r0 - R) & (cyv <= r0 + BANDH - 1 + R)
            g = ci * PCH + i * 16 + i16
            pos = jnp.where(m, lane_base + clv, jnp.full((16,), CAP,
                                                         jnp.int32) + i16)
            plsc.store_scatter(idx2, [pos], g)
            return clv + jnp.where(m, one16, z16i)

        return lax.fori_loop(0, PCH // 16, grp, clv0)

    clv = lax.fori_loop(0, NP // PCH, chunk_scan, z16i)

    def gath(r, carry):
        irow = idx2.at[pl.ds(r * 128, 128)]
        dst = pl.ds(r * 128, 128)
        pltpu.sync_copy(px_hbm.at[irow], pxb.at[dst])
        pltpu.sync_copy(py_hbm.at[irow], pyb.at[dst])
        pltpu.sync_copy(a_hbm.at[irow], ab.at[dst])
        pltpu.sync_copy(b_hbm.at[irow], bb.at[dst])
        pltpu.sync_copy(c_hbm.at[irow], cb.at[dst])
        pltpu.sync_copy(fr_hbm.at[irow], frb.at[dst])
        pltpu.sync_copy(fg_hbm.at[irow], fgb.at[dst])
        pltpu.sync_copy(fb_hbm.at[irow], fbb.at[dst])
        return carry

    lax.fori_loop(0, CAPR, gath, 0)

    def pstep(p, carry):
        gidx = lane_base + p
        pxg = plsc.load_gather(pxb, [gidx])
        pyg = plsc.load_gather(pyb, [gidx])
        ag = plsc.load_gather(ab, [gidx])
        bg = plsc.load_gather(bb, [gidx])
        cg = plsc.load_gather(cb, [gidx])
        frg = plsc.load_gather(frb, [gidx])
        fgg = plsc.load_gather(fgb, [gidx])
        fbg = plsc.load_gather(fbb, [gidx])
        valid = clv > p
        cxi = pxg.astype(jnp.int32)
        cyi = pyg.astype(jnp.int32)
        bx = cxi.astype(jnp.float32) + 0.5 - pxg
        halfA = 0.5 * ag
        halfC = 0.5 * cg
        z16 = z16f
        dxs = [bx + float(ox) for ox in range(-R, R + 1)]
        xins = [valid & (cxi + ox >= 0) & (cxi + ox < W)
                for ox in range(-R, R + 1)]
        xcls = [jnp.clip(cxi + ox, 0, W - 1) for ox in range(-R, R + 1)]
        cyb = cyi - R

        def row(rr, c3):
            ry = cyb + rr
            yok = (ry >= r0) & (ry < r0 + BANDH)
            dyv = ry.astype(jnp.float32) + 0.5 - pyg
            bdy = bg * dyv
            cterm = (halfC * dyv) * dyv
            ibase = jnp.clip(ry - r0, 0, BANDH - 1) * W
            for oxi in range(K):
                dx = dxs[oxi]
                sig = (halfA * dx) * dx + bdy * dx + cterm
                al = jnp.exp(-sig)
                m = xins[oxi] & yok & (sig >= 0.0)
                al = jnp.where(m, al, z16)
                idx = ibase + xcls[oxi]
                plsc.addupdate_scatter(im_r, [idx], al * frg)
                plsc.addupdate_scatter(im_g, [idx], al * fgg)
                plsc.addupdate_scatter(im_b, [idx], al * fbg)
            return c3

        lax.fori_loop(0, K, row, 0)
        return carry

    lax.fori_loop(0, LCAP, pstep, 0)

    def clipv(i, carry):
        for im in (im_r, im_g, im_b):
            im[pl.ds(i * 16, 16)] = jnp.clip(im[pl.ds(i * 16, 16)], 0.0, 1.0)
        return carry

    lax.fori_loop(0, FBN // 16, clipv, 0)
    for ch, im in enumerate((im_r, im_g, im_b)):
        def orow(yl, carry, im=im, ch=ch):
            pltpu.sync_copy(im.at[pl.ds(yl * W, W)],
                            out_hbm.at[ch, r0 + yl])
            return carry

        lax.fori_loop(0, BANDH, orow, 0)


def _band_scatter(px, py, a, b, c, fr, fg, fb):
    mesh = plsc.VectorSubcoreMesh(core_axis_name="c", subcore_axis_name="s")
    fn = pl.kernel(
        _band_body,
        out_type=jax.ShapeDtypeStruct((3, H, W), jnp.float32),
        mesh=mesh,
        scratch_types=[
            pltpu.VMEM((PCH,), jnp.float32),
            pltpu.VMEM((CAP + 16,), jnp.int32),
        ] + [pltpu.VMEM((CAP,), jnp.float32)] * 8 + [
            pltpu.VMEM((FBN,), jnp.float32),
            pltpu.VMEM((FBN,), jnp.float32),
            pltpu.VMEM((FBN,), jnp.float32),
        ],
        compiler_params=pltpu.CompilerParams(
            use_tc_tiling_on_sc=False, needs_layout_passes=False),
    )
    return fn(px, py, a, b, c, fr, fg, fb)


def _pad_inputs(xyz, scl, rot, feat, opac):
    p = NP - N
    t = (jnp.arange(p, dtype=jnp.float32) + 0.5) / p
    tx = jnp.arctanh(t * 1.98 - 0.99)
    ty = jnp.arctanh(((t * 37.0) % 1.0) * 1.98 - 0.99)
    xyz_p = jnp.concatenate([xyz, jnp.stack([tx, ty], axis=-1)], axis=0)
    scl_p = jnp.concatenate([scl, jnp.zeros((p, 2), jnp.float32)], axis=0)
    rot_p = jnp.concatenate([rot, jnp.zeros((p, 1), jnp.float32)], axis=0)
    feat_p = jnp.concatenate([feat, jnp.zeros((p, 3), jnp.float32)], axis=0)
    opac_p = jnp.concatenate([opac, jnp.zeros((p, 1), jnp.float32)], axis=0)
    return xyz_p, scl_p, rot_p, feat_p, opac_p


def kernel(_xyz, _scaling, _rotation, _features_dc, _opacity):
    xyz, scl, rot, feat, opac = _pad_inputs(
        _xyz, _scaling, _rotation, _features_dc, _opacity)
    px, py, a, b, c, fr, fg, fb = _prep_call(xyz, scl, rot, feat, opac)
    img = _band_scatter(px, py, a, b, c, fr, fg, fb)
    return img.reshape(1, 3, H, W)


# speedup vs baseline: 45.3937x; 2.2567x over previous
import functools

import jax
import jax.numpy as jnp
import numpy as np
from jax import lax
from jax.experimental import pallas as pl
from jax.experimental.pallas import tpu as pltpu
from jax.experimental.pallas import tpu_sc as plsc

N = 50000
H = 512
W = 512
R = 6
K = 2 * R + 1

G = 512
NP = 50176
NB = NP // G

PCH = 12544
LCAP = 296
CAP = 16 * LCAP
CAPR = CAP // 128
BANDH = 16
FBN = BANDH * W
GUARD = 8
FBA = FBN + 32


def _prep_body(xyz_ref, scl_ref, rot_ref, feat_ref, opac_ref,
               px_ref, py_ref, a_ref, b_ref, c_ref,
               fr_ref, fg_ref, fb_ref):
    xy = jnp.tanh(xyz_ref[...])
    px_ref[...] = (0.5 * W) * (xy[:, 0] + 1.0)
    py_ref[...] = (0.5 * H) * (xy[:, 1] + 1.0)
    scl = jnp.abs(scl_ref[...] + 0.5)
    s1 = scl[:, 0]
    s2 = scl[:, 1]
    rot = jax.nn.sigmoid(rot_ref[:, 0]) * (2.0 * np.pi)
    c = jnp.cos(rot)
    s = jnp.sin(rot)
    S00 = s1 * s1 * c * c + s2 * s2 * s * s
    S01 = (s1 * s1 - s2 * s2) * s * c
    S11 = s1 * s1 * s * s + s2 * s2 * c * c
    det = jnp.maximum(S00 * S11 - S01 * S01, 1e-12)
    a_ref[...] = S11 / det
    b_ref[...] = -S01 / det
    c_ref[...] = S00 / det
    f = feat_ref[...] * opac_ref[...]
    fr_ref[...] = f[:, 0]
    fg_ref[...] = f[:, 1]
    fb_ref[...] = f[:, 2]


def _prep_call(xyz, scl, rot, feat, opac):
    outs = [jax.ShapeDtypeStruct((NP,), jnp.float32)] * 8
    in_spec = lambda bs: pl.BlockSpec(bs, lambda i: (i, 0))
    out_spec = pl.BlockSpec((G,), lambda i: (i,))
    return pl.pallas_call(
        _prep_body,
        grid=(NB,),
        in_specs=[in_spec((G, 2)), in_spec((G, 2)), in_spec((G, 1)),
                  in_spec((G, 3)), in_spec((G, 1))],
        out_specs=[out_spec] * 8,
        out_shape=outs,
    )(xyz, scl, rot, feat, opac)


def _band_body(px_hbm, py_hbm, a_hbm, b_hbm, c_hbm, fr_hbm, fg_hbm, fb_hbm,
               out_hbm,
               pybuf, idx2, pxb, pyb, ab, bb, cb, frb, fgb, fbb,
               im_r, im_g, im_b):
    cidx = lax.axis_index("c")
    sidx = lax.axis_index("s")
    wid = cidx * 16 + sidx
    r0 = wid * BANDH
    i16 = lax.iota(jnp.int32, 16)
    z16f = jnp.zeros((16,), jnp.float32)
    z16i = jnp.zeros((16,), jnp.int32)

    def zero(i, carry):
        im_r[pl.ds(i * 16, 16)] = z16f
        im_g[pl.ds(i * 16, 16)] = z16f
        im_b[pl.ds(i * 16, 16)] = z16f
        return carry

    lax.fori_loop(0, FBA // 16, zero, 0)

    def prefill(i, carry):
        idx2[pl.ds(i * 16, 16)] = z16i
        return carry

    lax.fori_loop(0, CAP // 16, prefill, 0)

    one16 = jnp.full((16,), 1, jnp.int32)
    lane_base = i16 * LCAP

    def chunk_scan(ci, clv0):
        pltpu.sync_copy(py_hbm.at[pl.ds(ci * PCH, PCH)], pybuf)

        def grp(i, clv):
            pyv = pybuf[pl.ds(i * 16, 16)]
            cyv = pyv.astype(jnp.int32)
            m = (cyv >= r0 - R) & (cyv <= r0 + BANDH - 1 + R)
            g = ci * PCH + i * 16 + i16
            pos = jnp.where(m, lane_base + clv, jnp.full((16,), CAP,
                                                         jnp.int32) + i16)
            plsc.store_scatter(idx2, [pos], g)
            return clv + jnp.where(m, one16, z16i)

        return lax.fori_loop(0, PCH // 16, grp, clv0)

    clv = lax.fori_loop(0, NP // PCH, chunk_scan, z16i)

    def gath(r, carry):
        irow = idx2.at[pl.ds(r * 128, 128)]
        dst = pl.ds(r * 128, 128)
        pltpu.sync_copy(px_hbm.at[irow], pxb.at[dst])
        pltpu.sync_copy(py_hbm.at[irow], pyb.at[dst])
        pltpu.sync_copy(a_hbm.at[irow], ab.at[dst])
        pltpu.sync_copy(b_hbm.at[irow], bb.at[dst])
        pltpu.sync_copy(c_hbm.at[irow], cb.at[dst])
        pltpu.sync_copy(fr_hbm.at[irow], frb.at[dst])
        pltpu.sync_copy(fg_hbm.at[irow], fgb.at[dst])
        pltpu.sync_copy(fb_hbm.at[irow], fbb.at[dst])
        return carry

    lax.fori_loop(0, CAPR, gath, 0)

    lane13 = i16 <= (K - 1)

    for l in range(16):
        cnt_l = clv[l]

        def one_g(q, c2, l=l):
            slot = l * LCAP + q
            px1 = pxb[pl.ds(slot, 16)][0]
            py1 = pyb[pl.ds(slot, 16)][0]
            a1 = ab[pl.ds(slot, 16)][0]
            b1 = bb[pl.ds(slot, 16)][0]
            c1 = cb[pl.ds(slot, 16)][0]
            fr1 = frb[pl.ds(slot, 16)][0]
            fg1 = fgb[pl.ds(slot, 16)][0]
            fb1 = fbb[pl.ds(slot, 16)][0]
            cx1 = px1.astype(jnp.int32)
            cy1 = py1.astype(jnp.int32)
            xv = cx1 - R + i16
            dx = xv.astype(jnp.float32) + 0.5 - px1
            sigA = (0.5 * a1) * dx * dx
            bdx = b1 * dx
            halfC = 0.5 * c1
            xin = lane13 & (xv >= 0) & (xv < W)
            ibase0 = cx1 - R + GUARD + i16
            rlo = jnp.maximum(cy1 - R, r0)
            rhi = jnp.minimum(cy1 + R, r0 + BANDH - 1)

            def row(ry, c3):
                dy = ry.astype(jnp.float32) + 0.5 - py1
                sig = sigA + bdx * dy + (halfC * dy) * dy
                al = jnp.exp(-sig)
                m = xin & (sig >= 0.0)
                al = jnp.where(m, al, z16f)
                idx = (ry - r0) * W + ibase0
                plsc.addupdate_scatter(im_r, [idx], al * fr1)
                plsc.addupdate_scatter(im_g, [idx], al * fg1)
                plsc.addupdate_scatter(im_b, [idx], al * fb1)
                return c3

            lax.fori_loop(rlo, rhi + 1, row, c2)
            return c2

        lax.fori_loop(0, cnt_l, one_g, 0)

    def clipv(i, carry):
        for im in (im_r, im_g, im_b):
            im[pl.ds(i * 16, 16)] = jnp.clip(im[pl.ds(i * 16, 16)], 0.0, 1.0)
        return carry

    lax.fori_loop(0, FBA // 16, clipv, 0)
    for ch, im in enumerate((im_r, im_g, im_b)):
        def orow(yl, carry, im=im, ch=ch):
            pltpu.sync_copy(im.at[pl.ds(yl * W + GUARD, W)],
                            out_hbm.at[ch, r0 + yl])
            return carry

        lax.fori_loop(0, BANDH, orow, 0)


def _band_scatter(px, py, a, b, c, fr, fg, fb):
    mesh = plsc.VectorSubcoreMesh(core_axis_name="c", subcore_axis_name="s")
    fn = pl.kernel(
        _band_body,
        out_type=jax.ShapeDtypeStruct((3, H, W), jnp.float32),
        mesh=mesh,
        scratch_types=[
            pltpu.VMEM((PCH,), jnp.float32),
            pltpu.VMEM((CAP + 16,), jnp.int32),
        ] + [pltpu.VMEM((CAP + 16,), jnp.float32)] * 8 + [
            pltpu.VMEM((FBA,), jnp.float32),
            pltpu.VMEM((FBA,), jnp.float32),
            pltpu.VMEM((FBA,), jnp.float32),
        ],
        compiler_params=pltpu.CompilerParams(
            use_tc_tiling_on_sc=False, needs_layout_passes=False),
    )
    return fn(px, py, a, b, c, fr, fg, fb)


def _pad_inputs(xyz, scl, rot, feat, opac):
    p = NP - N
    t = (jnp.arange(p, dtype=jnp.float32) + 0.5) / p
    tx = jnp.arctanh(t * 1.98 - 0.99)
    ty = jnp.arctanh(((t * 37.0) % 1.0) * 1.98 - 0.99)
    xyz_p = jnp.concatenate([xyz, jnp.stack([tx, ty], axis=-1)], axis=0)
    scl_p = jnp.concatenate([scl, jnp.zeros((p, 2), jnp.float32)], axis=0)
    rot_p = jnp.concatenate([rot, jnp.zeros((p, 1), jnp.float32)], axis=0)
    feat_p = jnp.concatenate([feat, jnp.zeros((p, 3), jnp.float32)], axis=0)
    opac_p = jnp.concatenate([opac, jnp.zeros((p, 1), jnp.float32)], axis=0)
    return xyz_p, scl_p, rot_p, feat_p, opac_p


def kernel(_xyz, _scaling, _rotation, _features_dc, _opacity):
    xyz, scl, rot, feat, opac = _pad_inputs(
        _xyz, _scaling, _rotation, _features_dc, _opacity)
    px, py, a, b, c, fr, fg, fb = _prep_call(xyz, scl, rot, feat, opac)
    img = _band_scatter(px, py, a, b, c, fr, fg, fb)
    return img.reshape(1, 3, H, W)
